# baseline (speedup 1.0000x reference)
import jax
import jax.numpy as jnp
from jax import lax
from jax.experimental import pallas as pl
from jax.experimental.pallas import tpu as pltpu
from jax.experimental.pallas import tpu_sc as plsc

NC = 2
NS = 16
L = 16
NW = NC * NS

B = 16384
D = 64
SLAB = 8
BPW = B // NW
NSEM = 4


def _body(uid_hbm, iid_hbm, ut_hbm, it_hbm, out_hbm,
          uidx_v, iidx_v, u_rows, i_rows, out_v, *sems):
    usems = sems[:NSEM]
    isems = sems[NSEM:]
    wid = lax.axis_index("s") * NC + lax.axis_index("c")
    base = wid * BPW

    pltpu.sync_copy(uid_hbm.at[pl.ds(base, BPW)], uidx_v)
    pltpu.sync_copy(iid_hbm.at[pl.ds(base, BPW)], iidx_v)

    lanes = lax.iota(jnp.int32, L)
    perms = {h: lanes ^ h for h in (8, 4, 2, 1)}
    masks = {h: (lanes & h) != 0 for h in (8, 4, 2, 1)}

    def lperm(v, h):
        return v.at[perms[h]].get(mode="promise_in_bounds", unique_indices=True)

    PROWS = BPW // 2
    for pp in range(2):
        pbase = pp * PROWS
        NG = PROWS // L
        GQ = NG // NSEM

        for g in range(NG):
            q = g // GQ
            uvec = uidx_v[pl.ds(pbase + g * L, L)]
            ivec = iidx_v[pl.ds(pbase + g * L, L)]
            for rl in range(L):
                r = g * L + rl
                pltpu.async_copy(ut_hbm.at[pl.ds(uvec[rl], 1)],
                                 u_rows.at[pl.ds(r, 1)], usems[q])
                pltpu.async_copy(it_hbm.at[pl.ds(ivec[rl], 1)],
                                 i_rows.at[pl.ds(r, 1)], isems[q])

        chunk = PROWS // NSEM
        for j in range(NSEM):
            sl = pl.ds(j * chunk, chunk)
            pltpu.make_async_copy(ut_hbm.at[pl.ds(0, chunk)], u_rows.at[sl], usems[j]).wait()
            pltpu.make_async_copy(it_hbm.at[pl.ds(0, chunk)], i_rows.at[sl], isems[j]).wait()

        def group(g, _):
            vs = []
            for rl in range(L):
                r = g * L + rl
                acc = u_rows[r, pl.ds(0, L)] * i_rows[r, pl.ds(0, L)]
                for k in range(1, D // L):
                    acc += u_rows[r, pl.ds(k * L, L)] * i_rows[r, pl.ds(k * L, L)]
                vs.append(acc)
            for h in (8, 4, 2, 1):
                half = len(vs) // 2
                vs = [jnp.where(masks[h],
                                vs[q + half] + lperm(vs[q + half], h),
                                vs[q] + lperm(vs[q], h))
                      for q in range(half)]
            out_v[pl.ds(pbase + g * L, L)] = vs[0]
            return 0

        lax.fori_loop(0, BPW // L // 2, group, 0)

    pltpu.sync_copy(out_v, out_hbm.at[pl.ds(base, BPW)])


@jax.jit
def _mf_dot(user_id, item_id, user_table, item_table):
    mesh = plsc.VectorSubcoreMesh(core_axis_name="c", subcore_axis_name="s")
    return pl.kernel(
        _body,
        out_type=jax.ShapeDtypeStruct((B,), jnp.float32),
        mesh=mesh,
        scratch_types=[
            pltpu.VMEM((BPW,), jnp.int32),
            pltpu.VMEM((BPW,), jnp.int32),
            pltpu.VMEM((BPW // 2, D), jnp.float32),
            pltpu.VMEM((BPW // 2, D), jnp.float32),
            pltpu.VMEM((BPW,), jnp.float32),
        ] + [pltpu.SemaphoreType.DMA] * (2 * NSEM),
    )(user_id, item_id, user_table, item_table)


def kernel(user_id, item_id, user_table, item_table):
    return _mf_dot(user_id, item_id, user_table, item_table)
